# baseline (speedup 1.0000x reference)
import functools

import jax
import jax.numpy as jnp
from jax import lax
from jax.experimental import pallas as pl
from jax.experimental.pallas import tpu as pltpu
from jax.experimental.pallas import tpu_sc as plsc

N = 10000
NP = 10240
E = 320000
D_IN = 128
D_HID = 24
NC, NS = 2, 16
NW = NC * NS
CHUNK = 128
KJ = 80
EP = NW * KJ * CHUNK
RPT = NP // NS


def _mesh():
    return plsc.VectorSubcoreMesh(core_axis_name="c", subcore_axis_name="s",
                                  num_cores=NC, num_subcores=NS)


@functools.cache
def _make_degree_kernel():
    @functools.partial(
        pl.kernel,
        out_type=jax.ShapeDtypeStruct((NC, 2, NP), jnp.float32),
        mesh=_mesh(),
        scratch_types=[
            pltpu.VMEM((KJ, CHUNK), jnp.int32),
            pltpu.VMEM((KJ, CHUNK), jnp.int32),
            pltpu.VMEM((CHUNK,), jnp.float32),
            pltpu.VMEM_SHARED((NP,), jnp.float32),
            pltpu.VMEM_SHARED((NP,), jnp.float32),
            pltpu.SemaphoreType.DMA,
            pltpu.SemaphoreType.DMA,
        ],
        compiler_params=pltpu.CompilerParams(use_tc_tiling_on_sc=False),
    )
    def _degree_kernel(srcw, dstw, ones_hbm, zeros_hbm, out, src_v, dst_v,
                       ones_v, dego, degi, sem0, sem1):
        c = lax.axis_index("c")
        s = lax.axis_index("s")
        w = c * NS + s
        base = s * RPT
        pltpu.sync_copy(zeros_hbm.at[pl.ds(base, RPT)],
                        dego.at[pl.ds(base, RPT)])
        pltpu.sync_copy(zeros_hbm.at[pl.ds(base, RPT)],
                        degi.at[pl.ds(base, RPT)])
        pltpu.sync_copy(ones_hbm, ones_v)
        pltpu.sync_copy(srcw.at[w], src_v)
        pltpu.sync_copy(dstw.at[w], dst_v)
        plsc.subcore_barrier()

        @pl.loop(0, KJ)
        def _(j):
            pltpu.async_copy(ones_v, dego.at[src_v.at[j]], sem0, add=True)
            pltpu.async_copy(ones_v, degi.at[dst_v.at[j]], sem1, add=True)

        @pl.loop(0, KJ)
        def _(j):
            pltpu.make_async_copy(ones_v, dego.at[src_v.at[j]], sem0).wait()
            pltpu.make_async_copy(ones_v, degi.at[dst_v.at[j]], sem1).wait()

        plsc.subcore_barrier()
        pltpu.sync_copy(dego.at[pl.ds(base, RPT)],
                        out.at[c, 0, pl.ds(base, RPT)])
        pltpu.sync_copy(degi.at[pl.ds(base, RPT)],
                        out.at[c, 1, pl.ds(base, RPT)])

    return _degree_kernel


@functools.cache
def _make_conv_kernel():
    @functools.partial(
        pl.kernel,
        out_type=jax.ShapeDtypeStruct((NC, NP, D_HID), jnp.float32),
        mesh=_mesh(),
        scratch_types=[
            pltpu.VMEM((KJ, CHUNK), jnp.int32),
            pltpu.VMEM((KJ, CHUNK), jnp.int32),
            pltpu.VMEM((CHUNK, D_HID), jnp.float32),
            pltpu.VMEM((CHUNK, D_HID), jnp.float32),
            pltpu.VMEM((CHUNK, D_HID), jnp.float32),
            pltpu.VMEM((CHUNK, D_HID), jnp.float32),
            pltpu.VMEM_SHARED((NP, D_HID), jnp.float32),
            pltpu.VMEM_SHARED((NP, D_HID), jnp.float32),
            [pltpu.SemaphoreType.DMA] * 4,
            [pltpu.SemaphoreType.DMA] * 4,
        ],
        compiler_params=pltpu.CompilerParams(use_tc_tiling_on_sc=False),
    )
    def _conv_kernel(hw, srcw, dstw, zrows, out, src_v, dst_v, r0, r1, r2,
                     r3, accum, hw_sp, gsem, ssem):
        c = lax.axis_index("c")
        s = lax.axis_index("s")
        w = c * NS + s
        base = s * RPT
        pltpu.sync_copy(zrows.at[pl.ds(base, RPT)],
                        accum.at[pl.ds(base, RPT)])
        pltpu.sync_copy(hw.at[pl.ds(base, RPT)], hw_sp.at[pl.ds(base, RPT)])
        pltpu.sync_copy(srcw.at[w], src_v)
        pltpu.sync_copy(dstw.at[w], dst_v)
        plsc.subcore_barrier()

        rows = (r0, r1, r2, r3)
        for b in range(4):
            pltpu.async_copy(hw_sp.at[src_v.at[b]], rows[b], gsem[b])

        @pl.loop(0, KJ // 4)
        def _(q):
            j = 4 * q
            for b in range(4):
                pltpu.make_async_copy(hw_sp.at[src_v.at[j + b]], rows[b],
                                      gsem[b]).wait()
                pltpu.async_copy(rows[b], accum.at[dst_v.at[j + b]], ssem[b],
                                 add=True)
            for b in range(4):
                @pl.when(j + 4 + b < KJ)
                def _(b=b):
                    pltpu.make_async_copy(rows[b], accum.at[dst_v.at[j + b]],
                                          ssem[b]).wait()
                    pltpu.async_copy(hw_sp.at[src_v.at[j + 4 + b]], rows[b],
                                     gsem[b])

        for b in range(4):
            pltpu.make_async_copy(rows[b], accum.at[dst_v.at[KJ - 4 + b]],
                                  ssem[b]).wait()

        plsc.subcore_barrier()
        pltpu.sync_copy(accum.at[pl.ds(base, RPT)],
                        out.at[c, pl.ds(base, RPT)])

    return _conv_kernel


RB = 2560


def _tc_layer1(xp, W1, deg4):
    def body(x_ref, w_ref, d_ref, hw_ref, s_ref):
        d = d_ref[...]
        so = lax.rsqrt(jnp.maximum(d[:, 0:1] + d[:, 2:3], 1.0))
        si = lax.rsqrt(jnp.maximum(d[:, 1:2] + d[:, 3:4], 1.0))
        xw = jnp.dot(x_ref[...], w_ref[...],
                     preferred_element_type=jnp.float32)
        hw_ref[...] = xw * so
        s_ref[...] = jnp.concatenate([so, si], axis=1)

    return pl.pallas_call(
        body,
        grid=(NP // RB,),
        in_specs=[
            pl.BlockSpec((RB, D_IN), lambda i: (i, 0)),
            pl.BlockSpec((D_IN, D_HID), lambda i: (0, 0)),
            pl.BlockSpec((RB, 4), lambda i: (i, 0)),
        ],
        out_specs=[
            pl.BlockSpec((RB, D_HID), lambda i: (i, 0)),
            pl.BlockSpec((RB, 2), lambda i: (i, 0)),
        ],
        out_shape=[
            jax.ShapeDtypeStruct((NP, D_HID), jnp.float32),
            jax.ShapeDtypeStruct((NP, 2), jnp.float32),
        ],
    )(xp, W1, deg4)


def _tc_mid(a0, a1, S, b1, W2):
    def body(a0_ref, a1_ref, s_ref, b_ref, w_ref, o_ref):
        sv = s_ref[...]
        a = a0_ref[...] + a1_ref[...]
        h = jnp.maximum(a * sv[:, 1:2] + b_ref[...], 0.0)
        o_ref[...] = jnp.dot(h, w_ref[...],
                             preferred_element_type=jnp.float32) * sv[:, 0:1]

    return pl.pallas_call(
        body,
        grid=(NP // RB,),
        in_specs=[
            pl.BlockSpec((RB, D_HID), lambda i: (i, 0)),
            pl.BlockSpec((RB, D_HID), lambda i: (i, 0)),
            pl.BlockSpec((RB, 2), lambda i: (i, 0)),
            pl.BlockSpec((1, D_HID), lambda i: (0, 0)),
            pl.BlockSpec((D_HID, D_HID), lambda i: (0, 0)),
        ],
        out_specs=pl.BlockSpec((RB, D_HID), lambda i: (i, 0)),
        out_shape=jax.ShapeDtypeStruct((NP, D_HID), jnp.float32),
    )(a0, a1, S, b1, W2)


def _tc_post(a0, a1, S, b2):
    def body(a0_ref, a1_ref, s_ref, b_ref, o_ref):
        sv = s_ref[...]
        a = a0_ref[...] + a1_ref[...]
        o_ref[...] = jnp.maximum(a * sv[:, 1:2] + b_ref[...], 0.0)

    return pl.pallas_call(
        body,
        grid=(NP // RB,),
        in_specs=[
            pl.BlockSpec((RB, D_HID), lambda i: (i, 0)),
            pl.BlockSpec((RB, D_HID), lambda i: (i, 0)),
            pl.BlockSpec((RB, 2), lambda i: (i, 0)),
            pl.BlockSpec((1, D_HID), lambda i: (0, 0)),
        ],
        out_specs=pl.BlockSpec((RB, D_HID), lambda i: (i, 0)),
        out_shape=jax.ShapeDtypeStruct((NP, D_HID), jnp.float32),
    )(a0, a1, S, b2)


def _tc_head(xrp, WdP, bdP):
    def body(x_ref, w_ref, b_ref, o_ref):
        o_ref[...] = jnp.dot(x_ref[...], w_ref[...],
                             preferred_element_type=jnp.float32) + b_ref[...]

    return pl.pallas_call(
        body,
        in_specs=[
            pl.BlockSpec((2560, 4 * D_HID), lambda: (0, 0)),
            pl.BlockSpec((4 * D_HID, 8), lambda: (0, 0)),
            pl.BlockSpec((1, 8), lambda: (0, 0)),
        ],
        out_specs=pl.BlockSpec((2560, 8), lambda: (0, 0)),
        out_shape=jax.ShapeDtypeStruct((2560, 8), jnp.float32),
    )(xrp, WdP, bdP)


def kernel(x, edge_index, W1, b1, W2, b2, Wd, bd):
    f32 = jnp.float32
    src = edge_index[0].astype(jnp.int32)
    dst = edge_index[1].astype(jnp.int32)
    pad = EP - E
    src_t = jnp.concatenate([src, jnp.full((pad,), N, jnp.int32)]
                            ).reshape(NW, KJ, CHUNK)
    dst_t = jnp.concatenate([dst, jnp.full((pad,), N, jnp.int32)]
                            ).reshape(NW, KJ, CHUNK)
    ones128 = jnp.ones((CHUNK,), f32)
    zerosN = jnp.zeros((NP,), f32)
    zrows = jnp.zeros((NP, D_HID), f32)

    deg = _make_degree_kernel()(src_t, dst_t, ones128, zerosN)
    deg4 = deg.transpose(2, 0, 1).reshape(NP, 2 * NC)

    xp = jnp.pad(x, ((0, NP - N), (0, 0)))
    b1r = b1.reshape(1, D_HID)
    b2r = b2.reshape(1, D_HID)

    hw1, S = _tc_layer1(xp, W1, deg4)
    conv = _make_conv_kernel()
    agg1 = conv(hw1, src_t, dst_t, zrows)
    hw2 = _tc_mid(agg1[0], agg1[1], S, b1r, W2)
    agg2 = conv(hw2, src_t, dst_t, zrows)
    h2 = _tc_post(agg2[0], agg2[1], S, b2r)

    xr = h2[:N].reshape(N // 4, 4 * D_HID)
    xrp = jnp.pad(xr, ((0, 2560 - N // 4), (0, 0)))
    WdP = jnp.pad(Wd, ((0, 0), (0, 7)))
    bdP = jnp.pad(bd, (0, 7)).reshape(1, 8)
    out = _tc_head(xrp, WdP, bdP)
    return out[:N // 4, :1]


# speedup vs baseline: 1.0487x; 1.0487x over previous
import functools

import jax
import jax.numpy as jnp
from jax import lax
from jax.experimental import pallas as pl
from jax.experimental.pallas import tpu as pltpu
from jax.experimental.pallas import tpu_sc as plsc

N = 10000
NP = 10240
E = 320000
D_IN = 128
D_HID = 24
NC, NS = 2, 16
NW = NC * NS
CHUNK = 128
KJ = 80
EP = NW * KJ * CHUNK
RPT = NP // NS


def _mesh():
    return plsc.VectorSubcoreMesh(core_axis_name="c", subcore_axis_name="s",
                                  num_cores=NC, num_subcores=NS)


@functools.cache
def _make_degree_kernel():
    @functools.partial(
        pl.kernel,
        out_type=jax.ShapeDtypeStruct((NC, 2, NP), jnp.float32),
        mesh=_mesh(),
        scratch_types=[
            pltpu.VMEM((KJ, CHUNK), jnp.int32),
            pltpu.VMEM((KJ, CHUNK), jnp.int32),
            pltpu.VMEM((CHUNK,), jnp.float32),
            pltpu.VMEM_SHARED((NP,), jnp.float32),
            pltpu.VMEM_SHARED((NP,), jnp.float32),
            pltpu.SemaphoreType.DMA,
            pltpu.SemaphoreType.DMA,
        ],
        compiler_params=pltpu.CompilerParams(use_tc_tiling_on_sc=False),
    )
    def _degree_kernel(srcw, dstw, ones_hbm, zeros_hbm, out, src_v, dst_v,
                       ones_v, dego, degi, sem0, sem1):
        c = lax.axis_index("c")
        s = lax.axis_index("s")
        w = c * NS + s
        base = s * RPT
        pltpu.sync_copy(zeros_hbm.at[pl.ds(base, RPT)],
                        dego.at[pl.ds(base, RPT)])
        pltpu.sync_copy(zeros_hbm.at[pl.ds(base, RPT)],
                        degi.at[pl.ds(base, RPT)])
        pltpu.sync_copy(ones_hbm, ones_v)
        pltpu.sync_copy(srcw.at[w], src_v)
        pltpu.sync_copy(dstw.at[w], dst_v)
        plsc.subcore_barrier()

        @pl.loop(0, KJ)
        def _(j):
            a = pltpu.async_copy(ones_v, dego.at[src_v.at[j]], sem0,
                                 add=True)
            b = pltpu.async_copy(ones_v, degi.at[dst_v.at[j]], sem1,
                                 add=True)
            a.wait()
            b.wait()

        plsc.subcore_barrier()
        pltpu.sync_copy(dego.at[pl.ds(base, RPT)],
                        out.at[c, 0, pl.ds(base, RPT)])
        pltpu.sync_copy(degi.at[pl.ds(base, RPT)],
                        out.at[c, 1, pl.ds(base, RPT)])

    return _degree_kernel


@functools.cache
def _make_conv_kernel():
    @functools.partial(
        pl.kernel,
        out_type=jax.ShapeDtypeStruct((NC, NP, D_HID), jnp.float32),
        mesh=_mesh(),
        scratch_types=[
            pltpu.VMEM((KJ, CHUNK), jnp.int32),
            pltpu.VMEM((KJ, CHUNK), jnp.int32),
            pltpu.VMEM((CHUNK, D_HID), jnp.float32),
            pltpu.VMEM((CHUNK, D_HID), jnp.float32),
            pltpu.VMEM_SHARED((NP, D_HID), jnp.float32),
            pltpu.VMEM_SHARED((NP, D_HID), jnp.float32),
            pltpu.SemaphoreType.DMA,
            pltpu.SemaphoreType.DMA,
        ],
        compiler_params=pltpu.CompilerParams(use_tc_tiling_on_sc=False),
    )
    def _conv_kernel(hw, srcw, dstw, zrows, out, src_v, dst_v, rows0, rows1,
                     accum, hw_sp, sem0, sem1):
        c = lax.axis_index("c")
        s = lax.axis_index("s")
        w = c * NS + s
        base = s * RPT
        pltpu.sync_copy(zrows.at[pl.ds(base, RPT)],
                        accum.at[pl.ds(base, RPT)])
        pltpu.sync_copy(hw.at[pl.ds(base, RPT)], hw_sp.at[pl.ds(base, RPT)])
        pltpu.sync_copy(srcw.at[w], src_v)
        pltpu.sync_copy(dstw.at[w], dst_v)
        plsc.subcore_barrier()

        pltpu.async_copy(hw_sp.at[src_v.at[0]], rows0, sem0)

        @pl.loop(0, KJ // 2)
        def _(p):
            j = 2 * p
            pltpu.make_async_copy(hw_sp.at[src_v.at[j]], rows0, sem0).wait()
            pltpu.async_copy(hw_sp.at[src_v.at[j + 1]], rows1, sem1)
            pltpu.sync_copy(rows0, accum.at[dst_v.at[j]], add=True)

            @pl.when(j + 2 < KJ)
            def _():
                pltpu.async_copy(hw_sp.at[src_v.at[j + 2]], rows0, sem0)

            pltpu.make_async_copy(hw_sp.at[src_v.at[j + 1]], rows1,
                                  sem1).wait()
            pltpu.sync_copy(rows1, accum.at[dst_v.at[j + 1]], add=True)

        plsc.subcore_barrier()
        pltpu.sync_copy(accum.at[pl.ds(base, RPT)],
                        out.at[c, pl.ds(base, RPT)])

    return _conv_kernel


RB = 2560


def _tc_layer1(xp, W1, deg4):
    def body(x_ref, w_ref, d_ref, hw_ref, s_ref):
        d = d_ref[...]
        so = lax.rsqrt(jnp.maximum(d[:, 0:1] + d[:, 2:3], 1.0))
        si = lax.rsqrt(jnp.maximum(d[:, 1:2] + d[:, 3:4], 1.0))
        xw = jnp.dot(x_ref[...], w_ref[...],
                     preferred_element_type=jnp.float32)
        hw_ref[...] = xw * so
        s_ref[...] = jnp.concatenate([so, si], axis=1)

    return pl.pallas_call(
        body,
        grid=(NP // RB,),
        in_specs=[
            pl.BlockSpec((RB, D_IN), lambda i: (i, 0)),
            pl.BlockSpec((D_IN, D_HID), lambda i: (0, 0)),
            pl.BlockSpec((RB, 4), lambda i: (i, 0)),
        ],
        out_specs=[
            pl.BlockSpec((RB, D_HID), lambda i: (i, 0)),
            pl.BlockSpec((RB, 2), lambda i: (i, 0)),
        ],
        out_shape=[
            jax.ShapeDtypeStruct((NP, D_HID), jnp.float32),
            jax.ShapeDtypeStruct((NP, 2), jnp.float32),
        ],
    )(xp, W1, deg4)


def _tc_mid(a0, a1, S, b1, W2):
    def body(a0_ref, a1_ref, s_ref, b_ref, w_ref, o_ref):
        sv = s_ref[...]
        a = a0_ref[...] + a1_ref[...]
        h = jnp.maximum(a * sv[:, 1:2] + b_ref[...], 0.0)
        o_ref[...] = jnp.dot(h, w_ref[...],
                             preferred_element_type=jnp.float32) * sv[:, 0:1]

    return pl.pallas_call(
        body,
        grid=(NP // RB,),
        in_specs=[
            pl.BlockSpec((RB, D_HID), lambda i: (i, 0)),
            pl.BlockSpec((RB, D_HID), lambda i: (i, 0)),
            pl.BlockSpec((RB, 2), lambda i: (i, 0)),
            pl.BlockSpec((1, D_HID), lambda i: (0, 0)),
            pl.BlockSpec((D_HID, D_HID), lambda i: (0, 0)),
        ],
        out_specs=pl.BlockSpec((RB, D_HID), lambda i: (i, 0)),
        out_shape=jax.ShapeDtypeStruct((NP, D_HID), jnp.float32),
    )(a0, a1, S, b1, W2)


def _tc_post(a0, a1, S, b2):
    def body(a0_ref, a1_ref, s_ref, b_ref, o_ref):
        sv = s_ref[...]
        a = a0_ref[...] + a1_ref[...]
        o_ref[...] = jnp.maximum(a * sv[:, 1:2] + b_ref[...], 0.0)

    return pl.pallas_call(
        body,
        grid=(NP // RB,),
        in_specs=[
            pl.BlockSpec((RB, D_HID), lambda i: (i, 0)),
            pl.BlockSpec((RB, D_HID), lambda i: (i, 0)),
            pl.BlockSpec((RB, 2), lambda i: (i, 0)),
            pl.BlockSpec((1, D_HID), lambda i: (0, 0)),
        ],
        out_specs=pl.BlockSpec((RB, D_HID), lambda i: (i, 0)),
        out_shape=jax.ShapeDtypeStruct((NP, D_HID), jnp.float32),
    )(a0, a1, S, b2)


def _tc_head(xrp, WdP, bdP):
    def body(x_ref, w_ref, b_ref, o_ref):
        o_ref[...] = jnp.dot(x_ref[...], w_ref[...],
                             preferred_element_type=jnp.float32) + b_ref[...]

    return pl.pallas_call(
        body,
        in_specs=[
            pl.BlockSpec((2560, 4 * D_HID), lambda: (0, 0)),
            pl.BlockSpec((4 * D_HID, 8), lambda: (0, 0)),
            pl.BlockSpec((1, 8), lambda: (0, 0)),
        ],
        out_specs=pl.BlockSpec((2560, 8), lambda: (0, 0)),
        out_shape=jax.ShapeDtypeStruct((2560, 8), jnp.float32),
    )(xrp, WdP, bdP)


def kernel(x, edge_index, W1, b1, W2, b2, Wd, bd):
    f32 = jnp.float32
    src = edge_index[0].astype(jnp.int32)
    dst = edge_index[1].astype(jnp.int32)
    pad = EP - E
    src_t = jnp.concatenate([src, jnp.full((pad,), N, jnp.int32)]
                            ).reshape(NW, KJ, CHUNK)
    dst_t = jnp.concatenate([dst, jnp.full((pad,), N, jnp.int32)]
                            ).reshape(NW, KJ, CHUNK)
    ones128 = jnp.ones((CHUNK,), f32)
    zerosN = jnp.zeros((NP,), f32)
    zrows = jnp.zeros((NP, D_HID), f32)

    deg = _make_degree_kernel()(src_t, dst_t, ones128, zerosN)
    deg4 = deg.transpose(2, 0, 1).reshape(NP, 2 * NC)

    xp = jnp.pad(x, ((0, NP - N), (0, 0)))
    b1r = b1.reshape(1, D_HID)
    b2r = b2.reshape(1, D_HID)

    hw1, S = _tc_layer1(xp, W1, deg4)
    conv = _make_conv_kernel()
    agg1 = conv(hw1, src_t, dst_t, zrows)
    hw2 = _tc_mid(agg1[0], agg1[1], S, b1r, W2)
    agg2 = conv(hw2, src_t, dst_t, zrows)
    h2 = _tc_post(agg2[0], agg2[1], S, b2r)

    xr = h2[:N].reshape(N // 4, 4 * D_HID)
    xrp = jnp.pad(xr, ((0, 2560 - N // 4), (0, 0)))
    WdP = jnp.pad(Wd, ((0, 0), (0, 7)))
    bdP = jnp.pad(bd, (0, 7)).reshape(1, 8)
    out = _tc_head(xrp, WdP, bdP)
    return out[:N // 4, :1]
